# baseline (speedup 1.0000x reference)
import jax
import jax.numpy as jnp
from jax import lax
from jax.experimental import pallas as pl
from jax.experimental.pallas import tpu as pltpu
from jax.experimental.pallas import tpu_sc as plsc

_VOCAB = 100000
_D = 64
_B = 1024
_NC = 2
_NS = 16
_NW = _NC * _NS
_BPW = _B // _NW

_BN = 2048
_NSTEPS = -(-_VOCAB // _BN)
_NFULL = _NSTEPS - 1
_TAIL = _VOCAB - _NFULL * _BN
_NBUF = 4


def _gather_body(table_hbm, idx_hbm, out_hbm, idx_v, rows_v, sem):
    wid = lax.axis_index("s") * _NC + lax.axis_index("c")
    base = wid * _BPW
    pltpu.sync_copy(idx_hbm.at[pl.ds(base, _BPW)], idx_v)
    pltpu.async_copy(table_hbm.at[idx_v], rows_v, sem).wait()
    pltpu.sync_copy(rows_v, out_hbm.at[pl.ds(base, _BPW)])


def _make_gather():
    return pl.kernel(
        _gather_body,
        out_type=jax.ShapeDtypeStruct((_B, _D), jnp.float32),
        mesh=plsc.VectorSubcoreMesh(core_axis_name="c", subcore_axis_name="s"),
        scratch_types=[
            pltpu.VMEM((_BPW,), jnp.int32),
            pltpu.VMEM((_BPW, _D), jnp.float32),
            pltpu.SemaphoreType.DMA,
        ],
        compiler_params=pltpu.CompilerParams(use_tc_tiling_on_sc=False),
    )


def _matmul_body(u_ref, v_hbm, out_hbm, vbuf, vsem, tvbuf, obuf, osem, tobuf, tosem):
    i = pl.program_id(0)

    def vt_fetch(j):
        return pltpu.make_async_copy(
            v_hbm.at[pl.ds(j * _BN, _BN)], vbuf.at[lax.rem(j, 2)],
            vsem.at[lax.rem(j, 2)],
        )

    def tail_fetch():
        return pltpu.make_async_copy(
            v_hbm.at[pl.ds(_NFULL * _BN, _TAIL)], tvbuf, vsem.at[lax.rem(_NSTEPS - 1, 2)],
        )

    def out_push(j):
        return pltpu.make_async_copy(
            obuf.at[lax.rem(j, _NBUF)],
            out_hbm.at[:, pl.ds(j * _BN, _BN)],
            osem.at[lax.rem(j, _NBUF)],
        )

    def tail_push():
        return pltpu.make_async_copy(
            tobuf, out_hbm.at[:, pl.ds(_NFULL * _BN, _TAIL)], tosem,
        )

    @pl.when(i == 0)
    def _():
        vt_fetch(0).start()

    @pl.when(i < _NFULL - 1)
    def _():
        vt_fetch(i + 1).start()

    @pl.when(i == _NFULL - 1)
    def _():
        tail_fetch().start()

    @pl.when(jnp.logical_and(i >= _NBUF, i < _NFULL))
    def _():
        out_push(i - _NBUF).wait()

    @pl.when(i < _NFULL)
    def _():
        vt_fetch(i).wait()
        obuf[lax.rem(i, _NBUF)] = lax.dot_general(
            u_ref[...], vbuf[lax.rem(i, 2)],
            (((1,), (1,)), ((), ())),
            preferred_element_type=jnp.float32,
        )
        out_push(i).start()

    @pl.when(i == _NSTEPS - 1)
    def _():
        tail_fetch().wait()
        tobuf[...] = lax.dot_general(
            u_ref[...], tvbuf[...],
            (((1,), (1,)), ((), ())),
            preferred_element_type=jnp.float32,
        )
        tail_push().start()
        for k in range(_NBUF):
            out_push(_NFULL - _NBUF + k).wait()
        tail_push().wait()


def kernel(target_words, u_weight, v_weight):
    u_embeds = _make_gather()(u_weight, target_words.astype(jnp.int32))
    scores = pl.pallas_call(
        _matmul_body,
        grid=(_NSTEPS,),
        in_specs=[
            pl.BlockSpec((_B, _D), lambda i: (0, 0)),
            pl.BlockSpec(memory_space=pl.ANY),
        ],
        out_specs=pl.BlockSpec(memory_space=pl.ANY),
        out_shape=jax.ShapeDtypeStruct((_B, _VOCAB), jnp.float32),
        scratch_shapes=[
            pltpu.VMEM((2, _BN, _D), jnp.float32),
            pltpu.SemaphoreType.DMA((2,)),
            pltpu.VMEM((_TAIL, _D), jnp.float32),
            pltpu.VMEM((_NBUF, _B, _BN), jnp.float32),
            pltpu.SemaphoreType.DMA((_NBUF,)),
            pltpu.VMEM((_B, _TAIL), jnp.float32),
            pltpu.SemaphoreType.DMA,
        ],
        compiler_params=pltpu.CompilerParams(
            dimension_semantics=("arbitrary",),
        ),
    )(u_embeds, v_weight)
    return scores


# speedup vs baseline: 3.1660x; 3.1660x over previous
import jax
import jax.numpy as jnp
from jax import lax
from jax.experimental import pallas as pl
from jax.experimental.pallas import tpu as pltpu
from jax.experimental.pallas import tpu_sc as plsc

_VOCAB = 100000
_D = 64
_B = 1024
_NC = 2
_NS = 16
_NW = _NC * _NS
_DPW = _D // _NW
_NCHUNK = _B // 128

_BN = 2048
_NSTEPS = -(-_VOCAB // _BN)
_NFULL = _NSTEPS - 1
_TAIL = _VOCAB - _NFULL * _BN
_NBUF = 4


def _gather_body(flat_hbm, fidx_hbm, out_hbm, idx_v, rows_v, sem):
    wid = lax.axis_index("s") * _NC + lax.axis_index("c")
    d0 = wid * _DPW
    pltpu.sync_copy(fidx_hbm.at[pl.ds(d0, _DPW)], idx_v)
    copies = [
        pltpu.make_async_copy(
            flat_hbm.at[idx_v.at[r, c]],
            rows_v.at[r, pl.ds(c * 128, 128)],
            sem,
        )
        for r in range(_DPW)
        for c in range(_NCHUNK)
    ]
    for cp in copies:
        cp.start()
    for cp in copies:
        cp.wait()
    pltpu.sync_copy(rows_v, out_hbm.at[pl.ds(d0, _DPW)])


def _make_gather():
    return pl.kernel(
        _gather_body,
        out_type=jax.ShapeDtypeStruct((_D, _B), jnp.float32),
        mesh=plsc.VectorSubcoreMesh(core_axis_name="c", subcore_axis_name="s"),
        scratch_types=[
            pltpu.VMEM((_DPW, _NCHUNK, 128), jnp.int32),
            pltpu.VMEM((_DPW, _B), jnp.float32),
            pltpu.SemaphoreType.DMA,
        ],
        compiler_params=pltpu.CompilerParams(use_tc_tiling_on_sc=False),
    )


def _matmul_body(u_ref, vt_hbm, out_hbm, vbuf, vsem, tvbuf, obuf, osem, tobuf, tosem):
    i = pl.program_id(0)

    def vt_fetch(j):
        return pltpu.make_async_copy(
            vt_hbm.at[:, pl.ds(j * _BN, _BN)], vbuf.at[lax.rem(j, 2)],
            vsem.at[lax.rem(j, 2)],
        )

    def tail_fetch():
        return pltpu.make_async_copy(
            vt_hbm.at[:, pl.ds(_NFULL * _BN, _TAIL)], tvbuf,
            vsem.at[lax.rem(_NSTEPS - 1, 2)],
        )

    def out_push(j):
        return pltpu.make_async_copy(
            obuf.at[lax.rem(j, _NBUF)],
            out_hbm.at[pl.ds(j * _BN, _BN)],
            osem.at[lax.rem(j, _NBUF)],
        )

    def tail_push():
        return pltpu.make_async_copy(
            tobuf, out_hbm.at[pl.ds(_NFULL * _BN, _TAIL)], tosem,
        )

    @pl.when(i == 0)
    def _():
        vt_fetch(0).start()

    @pl.when(i < _NFULL - 1)
    def _():
        vt_fetch(i + 1).start()

    @pl.when(i == _NFULL - 1)
    def _():
        tail_fetch().start()

    @pl.when(jnp.logical_and(i >= _NBUF, i < _NFULL))
    def _():
        out_push(i - _NBUF).wait()

    @pl.when(i < _NFULL)
    def _():
        vt_fetch(i).wait()
        obuf[lax.rem(i, _NBUF)] = lax.dot_general(
            vbuf[lax.rem(i, 2)], u_ref[...],
            (((0,), (0,)), ((), ())),
            preferred_element_type=jnp.float32,
        )
        out_push(i).start()

    @pl.when(i == _NSTEPS - 1)
    def _():
        tail_fetch().wait()
        tobuf[...] = lax.dot_general(
            tvbuf[...], u_ref[...],
            (((0,), (0,)), ((), ())),
            preferred_element_type=jnp.float32,
        )
        tail_push().start()
        for k in range(_NBUF):
            out_push(_NFULL - _NBUF + k).wait()
        tail_push().wait()


def kernel(target_words, u_weight, v_weight):
    idx = target_words.astype(jnp.int32)
    fidx = (
        jnp.arange(_D, dtype=jnp.int32)[:, None] * _VOCAB + idx[None, :]
    ).reshape(_D, _NCHUNK, 128)
    ut_flat = u_weight.T.reshape(-1)
    ut_sel = _make_gather()(ut_flat, fidx)
    scores_t = pl.pallas_call(
        _matmul_body,
        grid=(_NSTEPS,),
        in_specs=[
            pl.BlockSpec((_D, _B), lambda i: (0, 0)),
            pl.BlockSpec(memory_space=pl.ANY),
        ],
        out_specs=pl.BlockSpec(memory_space=pl.ANY),
        out_shape=jax.ShapeDtypeStruct((_VOCAB, _B), jnp.float32),
        scratch_shapes=[
            pltpu.VMEM((2, _D, _BN), jnp.float32),
            pltpu.SemaphoreType.DMA((2,)),
            pltpu.VMEM((_D, _TAIL), jnp.float32),
            pltpu.VMEM((_NBUF, _BN, _B), jnp.float32),
            pltpu.SemaphoreType.DMA((_NBUF,)),
            pltpu.VMEM((_TAIL, _B), jnp.float32),
            pltpu.SemaphoreType.DMA,
        ],
        compiler_params=pltpu.CompilerParams(
            dimension_semantics=("arbitrary",),
        ),
    )(ut_sel, v_weight.T)
    return scores_t.T
